# baseline (speedup 1.0000x reference)
import functools
import jax
import jax.numpy as jnp
from jax import lax
from jax.experimental import pallas as pl
from jax.experimental.pallas import tpu as pltpu
from jax.experimental.pallas import tpu_sc as plsc

BATCH = 16384
HIST = 50
DIM = 32
B_TOTAL = BATCH * HIST
NC = 2
NS = 16
NW = NC * NS
B_PER_W = B_TOTAL // NW
CHUNK = 1024
NCHUNK = B_PER_W // CHUNK

_mesh = plsc.VectorSubcoreMesh(core_axis_name="c", subcore_axis_name="s")


@functools.partial(
    pl.kernel,
    mesh=_mesh,
    out_type=jax.ShapeDtypeStruct((B_TOTAL, DIM), jnp.bfloat16),
    scratch_types=[
        pltpu.VMEM((NCHUNK, CHUNK), jnp.int32),
        pltpu.VMEM((2, CHUNK, DIM), jnp.bfloat16),
        pltpu.SemaphoreType.DMA,
        pltpu.SemaphoreType.DMA,
    ],
    compiler_params=pltpu.CompilerParams(use_tc_tiling_on_sc=False),
)
def _emb_lookup(idx_hbm, table_hbm, out_hbm, idx_v, rows_v, gsem, ssem):
    wid = lax.axis_index("s") * NC + lax.axis_index("c")
    base = wid * B_PER_W
    pltpu.sync_copy(idx_hbm.at[wid], idx_v)

    def gather(g, b):
        return pltpu.make_async_copy(
            table_hbm.at[idx_v.at[g]], rows_v.at[b], gsem)

    def store(g, b):
        return pltpu.make_async_copy(
            rows_v.at[b], out_hbm.at[pl.ds(base + g * CHUNK, CHUNK)], ssem)

    gather(0, 0).start()

    def body(i, carry):
        cur = i % 2
        gather(i, cur).wait()
        gather(i + 1, 1 - cur).start()
        store(i, cur).start()
        store(i, cur).wait()
        return carry

    lax.fori_loop(0, NCHUNK - 1, body, 0)

    last = NCHUNK - 1
    gather(last, last % 2).wait()
    store(last, last % 2).start()
    store(last, last % 2).wait()


def kernel(input, table):
    idx = input.reshape(B_TOTAL).astype(jnp.int32).reshape(NW, NCHUNK, CHUNK)
    out = _emb_lookup(idx, table.astype(jnp.bfloat16))
    return out.astype(jnp.float32).reshape(BATCH, HIST, DIM)


# speedup vs baseline: 1.1227x; 1.1227x over previous
import functools
import jax
import jax.numpy as jnp
from jax import lax
from jax.experimental import pallas as pl
from jax.experimental.pallas import tpu as pltpu
from jax.experimental.pallas import tpu_sc as plsc

BATCH = 16384
HIST = 50
DIM = 32
B_TOTAL = BATCH * HIST
NC = 2
NS = 16
NW = NC * NS
B_PER_W = B_TOTAL // NW
CHUNK = 1024
NCHUNK = B_PER_W // CHUNK

_mesh = plsc.VectorSubcoreMesh(core_axis_name="c", subcore_axis_name="s")


@functools.partial(
    pl.kernel,
    mesh=_mesh,
    out_type=jax.ShapeDtypeStruct((B_TOTAL, DIM), jnp.float32),
    scratch_types=[
        pltpu.VMEM((B_PER_W,), jnp.int32),
        pltpu.VMEM((2, CHUNK, DIM), jnp.float32),
        pltpu.SemaphoreType.DMA,
        pltpu.SemaphoreType.DMA,
    ],
    compiler_params=pltpu.CompilerParams(use_tc_tiling_on_sc=False),
)
def _emb_lookup(idx_hbm, table_hbm, out_hbm, idx_v, rows_v, gsem, ssem):
    wid = lax.axis_index("s") * NC + lax.axis_index("c")
    base = wid * B_PER_W
    pltpu.sync_copy(idx_hbm.at[pl.ds(base, B_PER_W)], idx_v)

    def gather(g, b):
        return pltpu.make_async_copy(
            table_hbm.at[idx_v.at[pl.ds(g * CHUNK, CHUNK)]], rows_v.at[b], gsem)

    def store(g, b):
        return pltpu.make_async_copy(
            rows_v.at[b], out_hbm.at[pl.ds(base + g * CHUNK, CHUNK)], ssem)

    gather(0, 0).start()

    def body(i, carry):
        cur = i % 2
        gather(i, cur).wait()
        gather(i + 1, 1 - cur).start()
        store(i, cur).start()
        store(i, cur).wait()
        return carry

    lax.fori_loop(0, NCHUNK - 1, body, 0)

    last = NCHUNK - 1
    gather(last, last % 2).wait()
    store(last, last % 2).start()
    store(last, last % 2).wait()


def kernel(input, table):
    idx = input.reshape(B_TOTAL)
    out = _emb_lookup(idx, table)
    return out.reshape(BATCH, HIST, DIM)


# speedup vs baseline: 1.5186x; 1.3526x over previous
import functools
import jax
import jax.numpy as jnp
from jax import lax
from jax.experimental import pallas as pl
from jax.experimental.pallas import tpu as pltpu
from jax.experimental.pallas import tpu_sc as plsc

BATCH = 16384
HIST = 50
DIM = 32
B_TOTAL = BATCH * HIST
NC = 2
NS = 16
NW = NC * NS
BBLK = BATCH // NW
NPW = BBLK * HIST

_mesh = plsc.VectorSubcoreMesh(core_axis_name="c", subcore_axis_name="s")


@functools.partial(
    pl.kernel,
    mesh=_mesh,
    out_type=jax.ShapeDtypeStruct((HIST, 4, BATCH // 128, 8, 128), jnp.float32),
    scratch_types=[
        pltpu.VMEM((NPW,), jnp.int32),
        pltpu.VMEM((HIST, BBLK), jnp.int32),
        pltpu.VMEM((BBLK, DIM), jnp.float32),
        pltpu.VMEM((4, 4, 8, 128), jnp.float32),
        pltpu.SemaphoreType.DMA,
        pltpu.SemaphoreType.DMA,
    ],
    compiler_params=pltpu.CompilerParams(
        use_tc_tiling_on_sc=False, needs_layout_passes=False),
)
def _emb_lookup(idx_hbm, table_hbm, out_hbm, idx_v, idx_t, g_v, t_v, gsem, ssem):
    wid = lax.axis_index("s") * NC + lax.axis_index("c")
    base = wid * NPW
    pltpu.sync_copy(idx_hbm.at[pl.ds(base, NPW)], idx_v)

    lanes = lax.iota(jnp.int32, 16)
    step = lanes * HIST

    def build_t(h, carry):
        def inner(j, c2):
            ids = (j * 16) * HIST + h + step
            vals = plsc.load_gather(idx_v, [ids])
            idx_t[h, pl.ds(j * 16, 16)] = vals
            return c2
        lax.fori_loop(0, BBLK // 16, inner, 0)
        return carry

    lax.fori_loop(0, HIST, build_t, 0)

    def per_h(h, carry):
        g = pltpu.make_async_copy(table_hbm.at[idx_t.at[h]], g_v, gsem)
        g.start()
        g.wait()

        def per_rd(rd, c2):
            r = rd // 8
            dd = rd - r * 8
            for c in range(4):
                for j in range(8):
                    rows = c * 128 + j * 16 + lanes
                    cols = jnp.full((16,), dd, jnp.int32)
                    vals = plsc.load_gather(g_v, [rows, r * 8 + cols])
                    t_v[r, c, dd, pl.ds(j * 16, 16)] = vals
            return c2
        lax.fori_loop(0, 32, per_rd, 0)

        for r in range(4):
            pltpu.make_async_copy(
                t_v.at[r], out_hbm.at[h, r, pl.ds(wid * 4, 4)], ssem).start()
        for r in range(4):
            pltpu.make_async_copy(
                t_v.at[r], out_hbm.at[h, r, pl.ds(wid * 4, 4)], ssem).wait()
        return carry

    lax.fori_loop(0, HIST, per_h, 0)


def kernel(input, table):
    idx = input.reshape(B_TOTAL)
    out6 = _emb_lookup(idx, table)
    return out6.transpose(2, 4, 0, 1, 3).reshape(BATCH, HIST, DIM)


# speedup vs baseline: 1.6010x; 1.0543x over previous
import functools
import jax
import jax.numpy as jnp
from jax import lax
from jax.experimental import pallas as pl
from jax.experimental.pallas import tpu as pltpu
from jax.experimental.pallas import tpu_sc as plsc

BATCH = 16384
HIST = 50
DIM = 32
NC = 2
NS = 16
NW = NC * NS
BBLK = BATCH // NW

_mesh = plsc.VectorSubcoreMesh(core_axis_name="c", subcore_axis_name="s")


@functools.partial(
    pl.kernel,
    mesh=_mesh,
    out_type=jax.ShapeDtypeStruct((HIST, 4, BATCH // 128, 8, 128), jnp.float32),
    scratch_types=[
        pltpu.VMEM((BBLK, HIST), jnp.int32),
        pltpu.VMEM((HIST, BBLK), jnp.int32),
        pltpu.VMEM((2, BBLK, DIM), jnp.float32),
        pltpu.VMEM((2, 4, 4, 8, 128), jnp.float32),
        pltpu.SemaphoreType.DMA,
        pltpu.SemaphoreType.DMA,
    ],
    compiler_params=pltpu.CompilerParams(
        use_tc_tiling_on_sc=False, needs_layout_passes=False),
)
def _emb_lookup(idx_hbm, table_hbm, out_hbm, idx_v, idx_t, g_v, t_v, gsem, ssem):
    wid = lax.axis_index("s") * NC + lax.axis_index("c")
    b0 = wid * BBLK
    pltpu.sync_copy(idx_hbm.at[pl.ds(b0, BBLK)], idx_v)

    lanes = lax.iota(jnp.int32, 16)

    def build_t(h, carry):
        def inner(j, c2):
            vals = plsc.load_gather(
                idx_v, [j * 16 + lanes, jnp.full((16,), h, jnp.int32)])
            idx_t[h, pl.ds(j * 16, 16)] = vals
            return c2
        lax.fori_loop(0, BBLK // 16, inner, 0)
        return carry

    lax.fori_loop(0, HIST, build_t, 0)

    def gather(h, buf):
        return pltpu.make_async_copy(
            table_hbm.at[idx_t.at[h]], g_v.at[buf], gsem)

    def stores(h, buf):
        return [pltpu.make_async_copy(
            t_v.at[buf, r], out_hbm.at[h, r, pl.ds(wid * 4, 4)], ssem)
            for r in range(4)]

    def transpose(h, buf):
        def per_rd(rd, c2):
            r = lax.shift_right_logical(rd, 3)
            dd = lax.rem(rd, 8)
            cols = jnp.full((16,), rd, jnp.int32)
            for c in range(4):
                for j in range(8):
                    vals = plsc.load_gather(
                        g_v, [jnp.full((16,), buf, jnp.int32),
                              c * 128 + j * 16 + lanes, cols])
                    t_v[buf, r, c, dd, pl.ds(j * 16, 16)] = vals
            return c2
        lax.fori_loop(0, 32, per_rd, 0)

    g0 = gather(0, 0)
    g0.start()
    g0.wait()
    gather(1, 1).start()
    transpose(0, 0)
    for s in stores(0, 0):
        s.start()

    def body(h, carry):
        cur = h % 2
        gather(h, cur).wait()
        gather(h + 1, 1 - cur).start()
        for s in stores(h - 1, 1 - cur):
            s.wait()
        transpose(h, cur)
        for s in stores(h, cur):
            s.start()
        return carry

    lax.fori_loop(1, HIST - 1, body, 0)

    last = HIST - 1
    cur = last % 2
    gather(last, cur).wait()
    for s in stores(last - 1, 1 - cur):
        s.wait()
    transpose(last, cur)
    for s in stores(last, cur):
        s.start()
    for s in stores(last, cur):
        s.wait()


def kernel(input, table):
    out6 = _emb_lookup(input.astype(jnp.int32), table)
    return out6.transpose(2, 4, 0, 1, 3).reshape(BATCH, HIST, DIM)


# speedup vs baseline: 1.6151x; 1.0088x over previous
import functools
import jax
import jax.numpy as jnp
from jax import lax
from jax.experimental import pallas as pl
from jax.experimental.pallas import tpu as pltpu
from jax.experimental.pallas import tpu_sc as plsc

BATCH = 16384
HIST = 50
DIM = 32
NC = 2
NS = 16
NW = NC * NS
BBLK = BATCH // NW
HP = 56

_mesh = plsc.VectorSubcoreMesh(core_axis_name="c", subcore_axis_name="s")


@functools.partial(
    pl.kernel,
    mesh=_mesh,
    out_type=jax.ShapeDtypeStruct((HIST, 4, BATCH // 128, 8, 128), jnp.float32),
    scratch_types=[
        pltpu.VMEM((HP // 8, 4, 8, 128), jnp.int32),
        pltpu.VMEM((2, BBLK, DIM), jnp.float32),
        pltpu.VMEM((2, 4, 4, 8, 128), jnp.float32),
        pltpu.SemaphoreType.DMA,
        pltpu.SemaphoreType.DMA,
    ],
    compiler_params=pltpu.CompilerParams(
        use_tc_tiling_on_sc=False, needs_layout_passes=False),
)
def _emb_lookup(idx_hbm, table_hbm, out_hbm, idx_v, g_v, t_v, gsem, ssem):
    wid = lax.axis_index("s") * NC + lax.axis_index("c")
    for r in range(HP // 8):
        pltpu.sync_copy(idx_hbm.at[r, pl.ds(wid * 4, 4)], idx_v.at[r])

    lanes = lax.iota(jnp.int32, 16)

    def gathers(h, buf):
        r = lax.shift_right_logical(h, 3)
        hh = lax.rem(h, 8)
        return [pltpu.make_async_copy(
            table_hbm.at[idx_v.at[r, cl, hh]],
            g_v.at[buf, pl.ds(cl * 128, 128)], gsem)
            for cl in range(4)]

    def stores(h, buf):
        return [pltpu.make_async_copy(
            t_v.at[buf, r], out_hbm.at[h, r, pl.ds(wid * 4, 4)], ssem)
            for r in range(4)]

    def transpose(h, buf):
        bufv = jnp.full((16,), buf, jnp.int32)

        def per_rd(rd, c2):
            r = lax.shift_right_logical(rd, 3)
            dd = lax.rem(rd, 8)
            cols = jnp.full((16,), rd, jnp.int32)
            for c in range(4):
                for j in range(8):
                    vals = plsc.load_gather(
                        g_v, [bufv, c * 128 + j * 16 + lanes, cols])
                    t_v[buf, r, c, dd, pl.ds(j * 16, 16)] = vals
            return c2
        lax.fori_loop(0, 32, per_rd, 0)

    for g in gathers(0, 0):
        g.start()
    for g in gathers(0, 0):
        g.wait()
    for g in gathers(1, 1):
        g.start()
    transpose(0, 0)
    for s in stores(0, 0):
        s.start()

    def body(h, carry):
        cur = h % 2
        for g in gathers(h, cur):
            g.wait()
        for g in gathers(h + 1, 1 - cur):
            g.start()
        for s in stores(h - 1, 1 - cur):
            s.wait()
        transpose(h, cur)
        for s in stores(h, cur):
            s.start()
        return carry

    lax.fori_loop(1, HIST - 1, body, 0)

    last = HIST - 1
    cur = last % 2
    for g in gathers(last, cur):
        g.wait()
    for s in stores(last - 1, 1 - cur):
        s.wait()
    transpose(last, cur)
    for s in stores(last, cur):
        s.start()
    for s in stores(last, cur):
        s.wait()


def kernel(input, table):
    inp = jnp.pad(input.astype(jnp.int32), ((0, 0), (0, HP - HIST)))
    idx6 = inp.reshape(128, 128, HP // 8, 8).transpose(2, 0, 3, 1)
    out6 = _emb_lookup(idx6, table)
    return out6.transpose(2, 4, 0, 1, 3).reshape(BATCH, HIST, DIM)


# speedup vs baseline: 1.6222x; 1.0044x over previous
import functools
import jax
import jax.numpy as jnp
from jax import lax
from jax.experimental import pallas as pl
from jax.experimental.pallas import tpu as pltpu
from jax.experimental.pallas import tpu_sc as plsc

BATCH = 16384
HIST = 50
DIM = 32
NC = 2
NS = 16
NW = NC * NS
BBLK = BATCH // NW
HP = 56

_mesh = plsc.VectorSubcoreMesh(core_axis_name="c", subcore_axis_name="s")


@functools.partial(
    pl.kernel,
    mesh=_mesh,
    out_type=jax.ShapeDtypeStruct((HIST, 4, BATCH // 128, 8, 128), jnp.float32),
    scratch_types=[
        pltpu.VMEM((HP // 8, 4, 8, 128), jnp.int32),
        pltpu.VMEM((2, BBLK, DIM), jnp.float32),
        pltpu.VMEM((2, 4, 4, 8, 128), jnp.float32),
        pltpu.SemaphoreType.DMA,
        pltpu.SemaphoreType.DMA,
    ],
    compiler_params=pltpu.CompilerParams(
        use_tc_tiling_on_sc=False, needs_layout_passes=False),
)
def _emb_lookup(idx_hbm, table_hbm, out_hbm, idx_v, g_v, t_v, gsem, ssem):
    wid = lax.axis_index("s") * NC + lax.axis_index("c")
    for r in range(HP // 8):
        pltpu.sync_copy(idx_hbm.at[r, pl.ds(wid * 4, 4)], idx_v.at[r])

    lanes = lax.iota(jnp.int32, 16)

    def gathers(h, buf):
        r = lax.shift_right_logical(h, 3)
        hh = lax.rem(h, 8)
        return [pltpu.make_async_copy(
            table_hbm.at[idx_v.at[r, cl, hh]],
            g_v.at[buf, pl.ds(cl * 128, 128)], gsem)
            for cl in range(4)]

    def stores(h, buf):
        return [pltpu.make_async_copy(
            t_v.at[buf, r], out_hbm.at[h, r, pl.ds(wid * 4, 4)], ssem)
            for r in range(4)]

    def transpose(h, buf):
        g_b = g_v.at[buf]

        def per_rd(rd, c2):
            r = lax.shift_right_logical(rd, 3)
            dd = lax.rem(rd, 8)
            cols = jnp.full((16,), rd, jnp.int32)
            for c in range(4):
                for j in range(8):
                    vals = plsc.load_gather(
                        g_b, [c * 128 + j * 16 + lanes, cols])
                    t_v[buf, r, c, dd, pl.ds(j * 16, 16)] = vals
            return c2
        lax.fori_loop(0, 32, per_rd, 0)

    for g in gathers(0, 0):
        g.start()
    for g in gathers(0, 0):
        g.wait()
    for g in gathers(1, 1):
        g.start()
    transpose(0, 0)
    for s in stores(0, 0):
        s.start()

    def body(h, carry):
        cur = h % 2
        for g in gathers(h, cur):
            g.wait()
        for g in gathers(h + 1, 1 - cur):
            g.start()
        for s in stores(h - 1, 1 - cur):
            s.wait()
        transpose(h, cur)
        for s in stores(h, cur):
            s.start()
        return carry

    lax.fori_loop(1, HIST - 1, body, 0)

    last = HIST - 1
    cur = last % 2
    for g in gathers(last, cur):
        g.wait()
    for s in stores(last - 1, 1 - cur):
        s.wait()
    transpose(last, cur)
    for s in stores(last, cur):
        s.start()
    for s in stores(last, cur):
        s.wait()


def kernel(input, table):
    inp = jnp.pad(input.astype(jnp.int32), ((0, 0), (0, HP - HIST)))
    idx6 = inp.reshape(128, 128, HP // 8, 8).transpose(2, 0, 3, 1)
    out6 = _emb_lookup(idx6, table)
    return out6.transpose(2, 4, 0, 1, 3).reshape(BATCH, HIST, DIM)
